# baseline (speedup 1.0000x reference)
import jax
import jax.numpy as jnp
from jax.experimental import pallas as pl
from jax.experimental.pallas import tpu as pltpu

_N = 1024
_H = 256
_K = 8
_TR = 8


def _fused_kernel(nf_ref, w1_ref, b1_ref, w2_ref, b2_ref, ws1_ref, bs1_ref,
                  ws2_ref, bs2_ref, out_ref, lg_ref, wc_ref):
    x = nf_ref[0]

    s = jnp.maximum(
        jnp.dot(x, ws1_ref[...], preferred_element_type=jnp.float32) + bs1_ref[...],
        0.0,
    )
    bs2 = bs2_ref[0, 0]
    sc_col = jnp.dot(s, ws2_ref[...], preferred_element_type=jnp.float32) + bs2
    sc_row = jax.lax.dot_general(
        ws2_ref[...], s, (((0,), (1,)), ((), ())),
        preferred_element_type=jnp.float32,
    ) + bs2
    m = jnp.max(sc_row)
    e_row = jnp.exp(sc_row - m)
    inv_z = 1.0 / jnp.sum(e_row)
    w_row = e_row * inv_z
    wc_ref[...] = jnp.exp(sc_col - m) * inv_z

    h = jnp.maximum(
        jnp.dot(x, w1_ref[...], preferred_element_type=jnp.float32) + b1_ref[...],
        0.0,
    )
    lg_ref[...] = (
        jnp.dot(h, w2_ref[...], preferred_element_type=jnp.float32) + b2_ref[...]
    )

    ns = _N // 128

    def tile_body(t, _):
        r0 = t * _TR
        lg = lg_ref[pl.ds(r0, _TR), :]
        v = jax.nn.sigmoid(lg) * w_row
        wc = wc_ref[pl.ds(r0, _TR), :]
        sl = [v[:, k * 128:(k + 1) * 128] for k in range(ns)]
        for a, b in ((0, 1), (2, 3), (4, 5), (6, 7), (0, 2), (1, 3), (4, 6),
                     (5, 7), (1, 2), (5, 6), (0, 4), (3, 7), (1, 5), (2, 6),
                     (1, 4), (3, 6), (2, 4), (3, 5), (3, 4)):
            hi = jnp.maximum(sl[a], sl[b])
            lo = jnp.minimum(sl[a], sl[b])
            sl[a], sl[b] = hi, lo
        thresh = None
        for step in range(_K):
            thresh = jnp.max(sl[0], axis=1, keepdims=True)
            if step < _K - 1:
                msel = sl[0] == thresh
                for k in range(ns - 1):
                    sl[k] = jnp.where(msel, sl[k + 1], sl[k])
                sl[ns - 1] = jnp.where(msel, -1.0, sl[ns - 1])
        out_ref[0, pl.ds(r0, _TR), :] = jnp.where(v >= thresh, v * wc, 0.0)
        return 0

    jax.lax.fori_loop(0, _N // _TR, tile_body, 0)


def kernel(node_features, W1, b1, W2, b2, Ws1, bs1, Ws2, bs2):
    B, N, H = node_features.shape
    d32 = Ws1.shape[1]

    b1_2d = b1.reshape(1, H)
    b2_2d = b2.reshape(1, N)
    bs1_2d = bs1.reshape(1, d32)
    bs2_2d = bs2.reshape(1, 1)

    return pl.pallas_call(
        _fused_kernel,
        grid=(B,),
        in_specs=[
            pl.BlockSpec((1, N, H), lambda b: (b, 0, 0)),
            pl.BlockSpec((H, H), lambda b: (0, 0)),
            pl.BlockSpec((1, H), lambda b: (0, 0)),
            pl.BlockSpec((H, N), lambda b: (0, 0)),
            pl.BlockSpec((1, N), lambda b: (0, 0)),
            pl.BlockSpec((H, d32), lambda b: (0, 0)),
            pl.BlockSpec((1, d32), lambda b: (0, 0)),
            pl.BlockSpec((d32, 1), lambda b: (0, 0)),
            pl.BlockSpec((1, 1), lambda b: (0, 0)),
        ],
        out_specs=pl.BlockSpec((1, N, N), lambda b: (b, 0, 0)),
        out_shape=jax.ShapeDtypeStruct((B, N, N), jnp.float32),
        scratch_shapes=[
            pltpu.VMEM((N, N), jnp.float32),
            pltpu.VMEM((N, 1), jnp.float32),
        ],
    )(node_features, W1, b1_2d, W2, b2_2d, Ws1, bs1_2d, Ws2, bs2_2d)


# speedup vs baseline: 5.8969x; 5.8969x over previous
import jax
import jax.numpy as jnp
from jax.experimental import pallas as pl
from jax.experimental.pallas import tpu as pltpu

_N = 1024
_H = 256
_K = 8
_TR = 64


def _fused_kernel(nf_ref, w1_ref, b1_ref, w2_ref, b2_ref, ws1_ref, bs1_ref,
                  ws2_ref, bs2_ref, out_ref, lg_ref, wc_ref):
    x = nf_ref[0]

    s = jnp.maximum(
        jnp.dot(x, ws1_ref[...], preferred_element_type=jnp.float32) + bs1_ref[...],
        0.0,
    )
    bs2 = bs2_ref[0, 0]
    sc_col = jnp.dot(s, ws2_ref[...], preferred_element_type=jnp.float32) + bs2
    sc_row = jax.lax.dot_general(
        ws2_ref[...], s, (((0,), (1,)), ((), ())),
        preferred_element_type=jnp.float32,
    ) + bs2
    m = jnp.max(sc_row)
    e_row = jnp.exp(sc_row - m)
    inv_z = 1.0 / jnp.sum(e_row)
    w_row = e_row * inv_z
    wc_ref[...] = jnp.exp(sc_col - m) * inv_z

    h = jnp.maximum(
        jnp.dot(x, w1_ref[...], preferred_element_type=jnp.float32) + b1_ref[...],
        0.0,
    )
    lg_ref[...] = (
        jnp.dot(h, w2_ref[...], preferred_element_type=jnp.float32) + b2_ref[...]
    )

    ns = _N // 128

    def tile_body(t, _):
        r0 = t * _TR
        lg = lg_ref[pl.ds(r0, _TR), :]
        v = jax.nn.sigmoid(lg) * w_row
        wc = wc_ref[pl.ds(r0, _TR), :]
        sl = [v[:, k * 128:(k + 1) * 128] for k in range(ns)]
        for a, b in ((0, 1), (2, 3), (4, 5), (6, 7), (0, 2), (1, 3), (4, 6),
                     (5, 7), (1, 2), (5, 6), (0, 4), (3, 7), (1, 5), (2, 6),
                     (1, 4), (3, 6), (2, 4), (3, 5), (3, 4)):
            hi = jnp.maximum(sl[a], sl[b])
            lo = jnp.minimum(sl[a], sl[b])
            sl[a], sl[b] = hi, lo
        thresh = None
        for step in range(_K):
            thresh = jnp.max(sl[0], axis=1, keepdims=True)
            if step < _K - 1:
                msel = sl[0] == thresh
                for k in range(ns - 1):
                    sl[k] = jnp.where(msel, sl[k + 1], sl[k])
                sl[ns - 1] = jnp.where(msel, -1.0, sl[ns - 1])
        out_ref[0, pl.ds(r0, _TR), :] = jnp.where(v >= thresh, v * wc, 0.0)
        return 0

    jax.lax.fori_loop(0, _N // _TR, tile_body, 0)


def kernel(node_features, W1, b1, W2, b2, Ws1, bs1, Ws2, bs2):
    B, N, H = node_features.shape
    d32 = Ws1.shape[1]

    b1_2d = b1.reshape(1, H)
    b2_2d = b2.reshape(1, N)
    bs1_2d = bs1.reshape(1, d32)
    bs2_2d = bs2.reshape(1, 1)

    return pl.pallas_call(
        _fused_kernel,
        grid=(B,),
        in_specs=[
            pl.BlockSpec((1, N, H), lambda b: (b, 0, 0)),
            pl.BlockSpec((H, H), lambda b: (0, 0)),
            pl.BlockSpec((1, H), lambda b: (0, 0)),
            pl.BlockSpec((H, N), lambda b: (0, 0)),
            pl.BlockSpec((1, N), lambda b: (0, 0)),
            pl.BlockSpec((H, d32), lambda b: (0, 0)),
            pl.BlockSpec((1, d32), lambda b: (0, 0)),
            pl.BlockSpec((d32, 1), lambda b: (0, 0)),
            pl.BlockSpec((1, 1), lambda b: (0, 0)),
        ],
        out_specs=pl.BlockSpec((1, N, N), lambda b: (b, 0, 0)),
        out_shape=jax.ShapeDtypeStruct((B, N, N), jnp.float32),
        scratch_shapes=[
            pltpu.VMEM((N, N), jnp.float32),
            pltpu.VMEM((N, 1), jnp.float32),
        ],
    )(node_features, W1, b1_2d, W2, b2_2d, Ws1, bs1_2d, Ws2, bs2_2d)


# speedup vs baseline: 14.3540x; 2.4342x over previous
import jax
import jax.numpy as jnp
from jax.experimental import pallas as pl

_N = 1024
_H = 256
_K = 8


def _fused_kernel(nf_ref, w1_ref, b1_ref, w2_ref, b2_ref, ws1_ref, bs1_ref,
                  ws2_ref, bs2_ref, out_ref):
    x = nf_ref[0]

    s = jnp.maximum(
        jnp.dot(x, ws1_ref[...], preferred_element_type=jnp.float32) + bs1_ref[...],
        0.0,
    )
    bs2 = bs2_ref[0, 0]
    sc_col = jnp.dot(s, ws2_ref[...], preferred_element_type=jnp.float32) + bs2
    sc_row = jax.lax.dot_general(
        ws2_ref[...], s, (((0,), (1,)), ((), ())),
        preferred_element_type=jnp.float32,
    ) + bs2
    m = jnp.max(sc_row)
    e_row = jnp.exp(sc_row - m)
    inv_z = 1.0 / jnp.sum(e_row)
    w_row = e_row * inv_z
    w_col = jnp.exp(sc_col - m) * inv_z

    h = jnp.maximum(
        jnp.dot(x, w1_ref[...], preferred_element_type=jnp.float32) + b1_ref[...],
        0.0,
    )
    logits = jnp.dot(h, w2_ref[...], preferred_element_type=jnp.float32) + b2_ref[...]
    v = jax.nn.sigmoid(logits) * w_row

    ns = _N // 128
    sl = [v[:, k * 128:(k + 1) * 128] for k in range(ns)]
    for a, b in ((0, 1), (2, 3), (4, 5), (6, 7), (0, 2), (1, 3), (4, 6),
                 (5, 7), (1, 2), (5, 6), (0, 4), (3, 7), (1, 5), (2, 6),
                 (1, 4), (3, 6), (2, 4), (3, 5), (3, 4)):
        hi = jnp.maximum(sl[a], sl[b])
        lo = jnp.minimum(sl[a], sl[b])
        sl[a], sl[b] = hi, lo
    thresh = None
    for step in range(_K):
        thresh = jnp.max(sl[0], axis=1, keepdims=True)
        if step < _K - 1:
            msel = sl[0] == thresh
            for k in range(ns - 1 - step):
                sl[k] = jnp.where(msel, sl[k + 1], sl[k])
    out_ref[0] = jnp.where(v >= thresh, v * w_col, 0.0)


def kernel(node_features, W1, b1, W2, b2, Ws1, bs1, Ws2, bs2):
    B, N, H = node_features.shape
    d32 = Ws1.shape[1]

    b1_2d = b1.reshape(1, H)
    b2_2d = b2.reshape(1, N)
    bs1_2d = bs1.reshape(1, d32)
    bs2_2d = bs2.reshape(1, 1)

    return pl.pallas_call(
        _fused_kernel,
        grid=(B,),
        in_specs=[
            pl.BlockSpec((1, N, H), lambda b: (b, 0, 0)),
            pl.BlockSpec((H, H), lambda b: (0, 0)),
            pl.BlockSpec((1, H), lambda b: (0, 0)),
            pl.BlockSpec((H, N), lambda b: (0, 0)),
            pl.BlockSpec((1, N), lambda b: (0, 0)),
            pl.BlockSpec((H, d32), lambda b: (0, 0)),
            pl.BlockSpec((1, d32), lambda b: (0, 0)),
            pl.BlockSpec((d32, 1), lambda b: (0, 0)),
            pl.BlockSpec((1, 1), lambda b: (0, 0)),
        ],
        out_specs=pl.BlockSpec((1, N, N), lambda b: (b, 0, 0)),
        out_shape=jax.ShapeDtypeStruct((B, N, N), jnp.float32),
    )(node_features, W1, b1_2d, W2, b2_2d, Ws1, bs1_2d, Ws2, bs2_2d)


# speedup vs baseline: 14.9259x; 1.0398x over previous
import jax
import jax.numpy as jnp
from jax.experimental import pallas as pl

_N = 1024
_H = 256
_K = 8


def _fused_kernel(nf_ref, w1_ref, b1_ref, w2_ref, b2_ref, ws1_ref, bs1_ref,
                  ws2_ref, bs2_ref, out_ref):
    x = nf_ref[0]

    s = jnp.maximum(
        jnp.dot(x, ws1_ref[...], preferred_element_type=jnp.float32) + bs1_ref[...],
        0.0,
    )
    bs2 = bs2_ref[0, 0]
    sc_col = jnp.dot(s, ws2_ref[...], preferred_element_type=jnp.float32) + bs2
    sc_row = jax.lax.dot_general(
        ws2_ref[...], s, (((0,), (1,)), ((), ())),
        preferred_element_type=jnp.float32,
    ) + bs2
    m = jnp.max(sc_row)
    e_row = jnp.exp(sc_row - m)
    inv_z = 1.0 / jnp.sum(e_row)
    w_row = e_row * inv_z
    w_col = jnp.exp(sc_col - m) * inv_z

    h = jnp.maximum(
        jnp.dot(x, w1_ref[...], preferred_element_type=jnp.float32) + b1_ref[...],
        0.0,
    )
    logits = jnp.dot(h, w2_ref[...], preferred_element_type=jnp.float32) + b2_ref[...]
    v = jax.nn.sigmoid(logits) * w_row * w_col

    ns = _N // 128
    sl = [v[:, k * 128:(k + 1) * 128] for k in range(ns)]
    for a, b in ((0, 1), (2, 3), (4, 5), (6, 7), (0, 2), (1, 3), (4, 6),
                 (5, 7), (1, 2), (5, 6), (0, 4), (3, 7), (1, 5), (2, 6),
                 (1, 4), (3, 6), (2, 4), (3, 5), (3, 4)):
        hi = jnp.maximum(sl[a], sl[b])
        lo = jnp.minimum(sl[a], sl[b])
        sl[a], sl[b] = hi, lo
    thresh = None
    for step in range(_K):
        thresh = jnp.max(sl[0], axis=1, keepdims=True)
        if step < _K - 1:
            msel = sl[0] == thresh
            for k in range(ns - 1 - step):
                sl[k] = jnp.where(msel, sl[k + 1], sl[k])
    out_ref[0] = jnp.where(v >= thresh, v, 0.0)


def kernel(node_features, W1, b1, W2, b2, Ws1, bs1, Ws2, bs2):
    B, N, H = node_features.shape
    d32 = Ws1.shape[1]

    b1_2d = b1.reshape(1, H)
    b2_2d = b2.reshape(1, N)
    bs1_2d = bs1.reshape(1, d32)
    bs2_2d = bs2.reshape(1, 1)

    return pl.pallas_call(
        _fused_kernel,
        grid=(B,),
        in_specs=[
            pl.BlockSpec((1, N, H), lambda b: (b, 0, 0)),
            pl.BlockSpec((H, H), lambda b: (0, 0)),
            pl.BlockSpec((1, H), lambda b: (0, 0)),
            pl.BlockSpec((H, N), lambda b: (0, 0)),
            pl.BlockSpec((1, N), lambda b: (0, 0)),
            pl.BlockSpec((H, d32), lambda b: (0, 0)),
            pl.BlockSpec((1, d32), lambda b: (0, 0)),
            pl.BlockSpec((d32, 1), lambda b: (0, 0)),
            pl.BlockSpec((1, 1), lambda b: (0, 0)),
        ],
        out_specs=pl.BlockSpec((1, N, N), lambda b: (b, 0, 0)),
        out_shape=jax.ShapeDtypeStruct((B, N, N), jnp.float32),
    )(node_features, W1, b1_2d, W2, b2_2d, Ws1, bs1_2d, Ws2, bs2_2d)
